# Initial kernel scaffold; baseline (speedup 1.0000x reference)
import jax
import jax.numpy as jnp
from jax.experimental import pallas as pl


def kernel(input_ids, speaker_ids, char_emb, pos_table, speaker_emb, fc_W, fc_b):
    raise NotImplementedError("write your pallas kernel here")


# speedup vs baseline: 2.7880x; 2.7880x over previous
import functools

import jax
import jax.numpy as jnp
from jax import lax
from jax.experimental import pallas as pl
from jax.experimental.pallas import tpu as pltpu
from jax.experimental.pallas import tpu_sc as plsc

H = 128
SEQ = 200
BATCH = 1024
NSPK = 10
N = BATCH * SEQ
NC, NS = 2, 16
NW = NC * NS
PER_W = N // NW
C = 128
NCHUNK = PER_W // C


def _tc_combo(speaker_ids2d, pos_rows, speaker_emb, fc_W, fc_b2d):
    def body(spk_ref, pos_ref, semb_ref, w_ref, b_ref, combo_ref, bidx_ref):
        x = jnp.dot(semb_ref[...], w_ref[...],
                    preferred_element_type=jnp.float32) + b_ref[...]
        feat = jnp.maximum(x, 0.0) + jnp.log1p(jnp.exp(-jnp.abs(x)))
        combo_ref[...] = feat[:, None, :] + pos_ref[...][None, :, :]
        bidx_ref[...] = spk_ref[...] * SEQ + lax.broadcasted_iota(
            jnp.int32, (BATCH, SEQ), 1)

    return pl.pallas_call(
        body,
        out_shape=(
            jax.ShapeDtypeStruct((NSPK, SEQ, H), jnp.float32),
            jax.ShapeDtypeStruct((BATCH, SEQ), jnp.int32),
        ),
    )(speaker_ids2d, pos_rows, speaker_emb, fc_W, fc_b2d)


def _sc_gather(char_emb, ids_flat, bidx_flat, combo2d):
    mesh = plsc.VectorSubcoreMesh(core_axis_name="c", subcore_axis_name="s")

    @functools.partial(
        pl.kernel,
        out_type=jax.ShapeDtypeStruct((N, H), jnp.float32),
        mesh=mesh,
        scratch_types=[
            pltpu.VMEM((C,), jnp.int32),
            pltpu.VMEM((C,), jnp.int32),
            pltpu.VMEM((C, H), jnp.float32),
            pltpu.VMEM((C, H), jnp.float32),
            pltpu.SemaphoreType.DMA,
            pltpu.SemaphoreType.DMA,
        ],
    )
    def k(char_hbm, ids_hbm, bidx_hbm, combo_hbm, out_hbm,
          idx_v, bidx_v, rows_v, crows_v, sem1, sem2):
        wid = lax.axis_index("s") * NC + lax.axis_index("c")
        base_w = wid * PER_W

        def chunk(c, carry):
            base = base_w + c * C
            pltpu.sync_copy(ids_hbm.at[pl.ds(base, C)], idx_v)
            pltpu.sync_copy(bidx_hbm.at[pl.ds(base, C)], bidx_v)
            cp1 = pltpu.async_copy(char_hbm.at[idx_v], rows_v, sem1)
            cp2 = pltpu.async_copy(combo_hbm.at[bidx_v], crows_v, sem2)
            cp1.wait()
            cp2.wait()

            def row(r, cc):
                for j in range(H // 16):
                    plsc.addupdate(rows_v.at[r, pl.ds(j * 16, 16)],
                                   crows_v[r, pl.ds(j * 16, 16)])
                return cc

            lax.fori_loop(0, C, row, 0)
            pltpu.sync_copy(rows_v, out_hbm.at[pl.ds(base, C)])
            return carry

        lax.fori_loop(0, NCHUNK, chunk, 0)

    return k(char_emb, ids_flat, bidx_flat, combo2d)


def kernel(input_ids, speaker_ids, char_emb, pos_table, speaker_emb, fc_W, fc_b):
    pos_rows = lax.slice_in_dim(pos_table, 1, SEQ + 1, axis=0)
    combo3, bidx = _tc_combo(speaker_ids[:, None].astype(jnp.int32), pos_rows,
                             speaker_emb, fc_W, fc_b[None, :])
    out = _sc_gather(char_emb, input_ids.reshape(N), bidx.reshape(N),
                     combo3.reshape(NSPK * SEQ, H))
    return out.reshape(BATCH, SEQ, H)
